# baseline (speedup 1.0000x reference)
import functools

import jax
import jax.numpy as jnp
from jax import lax
from jax.experimental import pallas as pl
from jax.experimental.pallas import tpu as pltpu
from jax.experimental.pallas import tpu_sc as plsc

_BATCH, _SEQ, _D = 4096, 200, 64
_B = _BATCH * _SEQ
_NC, _NS = 2, 16
_NW = _NC * _NS
_RPW = _B // _NW
_K = 128
_G = _RPW // _K
_CH = 512
_GPC = _CH // _K
_NCH = _RPW // _CH


def _build():
    mesh = plsc.VectorSubcoreMesh(core_axis_name="c", subcore_axis_name="s")

    @functools.partial(
        pl.kernel,
        mesh=mesh,
        out_type=jax.ShapeDtypeStruct((_B, _D), jnp.float32),
        scratch_types=[
            pltpu.VMEM((_G, _K), jnp.int32),
            pltpu.VMEM((_CH, _D), jnp.float32),
            pltpu.SemaphoreType.DMA,
        ],
        compiler_params=pltpu.CompilerParams(use_tc_tiling_on_sc=False),
    )
    def emb(idx_hbm, table_hbm, out_hbm, idx_v, buf, gsem):
        wid = lax.axis_index("s") * _NC + lax.axis_index("c")
        row0 = wid * _RPW
        pltpu.sync_copy(idx_hbm.at[wid], idx_v)

        def chunk_body(c, carry):
            handles = []
            for g in range(_GPC):
                handles.append(
                    pltpu.async_copy(
                        table_hbm.at[idx_v.at[c * _GPC + g]],
                        buf.at[pl.ds(g * _K, _K)],
                        gsem,
                    )
                )
            for h in handles:
                h.wait()
            pltpu.sync_copy(buf, out_hbm.at[pl.ds(row0 + c * _CH, _CH)])
            return carry

        lax.fori_loop(0, _NCH, chunk_body, 0)

    return emb


_emb = _build()


@jax.jit
def kernel(input_texts, embedding_table):
    idx = input_texts.reshape(_NW, _G, _K)
    out = _emb(idx, embedding_table)
    return out.reshape(_BATCH, _SEQ, _D)


# speedup vs baseline: 1.0192x; 1.0192x over previous
import functools

import jax
import jax.numpy as jnp
from jax import lax
from jax.experimental import pallas as pl
from jax.experimental.pallas import tpu as pltpu
from jax.experimental.pallas import tpu_sc as plsc

_BATCH, _SEQ, _D = 4096, 200, 64
_B = _BATCH * _SEQ
_NC, _NS = 2, 16
_NW = _NC * _NS
_RPW = _B // _NW
_K = 128
_G = _RPW // _K
_CH = 512
_GPC = _CH // _K
_NCH = _RPW // _CH


def _build():
    mesh = plsc.VectorSubcoreMesh(core_axis_name="c", subcore_axis_name="s")

    @functools.partial(
        pl.kernel,
        mesh=mesh,
        out_type=jax.ShapeDtypeStruct((_B, _D), jnp.float32),
        scratch_types=[
            pltpu.VMEM((_G, _K), jnp.int32),
            pltpu.VMEM((2, _CH, _D), jnp.float32),
            pltpu.SemaphoreType.DMA,
            pltpu.SemaphoreType.DMA,
            pltpu.SemaphoreType.DMA,
            pltpu.SemaphoreType.DMA,
        ],
        compiler_params=pltpu.CompilerParams(use_tc_tiling_on_sc=False),
    )
    def emb(idx_hbm, table_hbm, out_hbm, idx_v, buf, gsem0, gsem1, osem0, osem1):
        wid = lax.axis_index("s") * _NC + lax.axis_index("c")
        row0 = wid * _RPW
        pltpu.sync_copy(idx_hbm.at[wid], idx_v)
        gsems = (gsem0, gsem1)
        osems = (osem0, osem1)

        def start_gather(c, s):
            return [
                pltpu.async_copy(
                    table_hbm.at[idx_v.at[c * _GPC + g]],
                    buf.at[s, pl.ds(g * _K, _K)],
                    gsems[s],
                )
                for g in range(_GPC)
            ]

        def start_out(c, s):
            pltpu.async_copy(
                buf.at[s], out_hbm.at[pl.ds(row0 + c * _CH, _CH)], osems[s]
            )

        def wait_out(c, s):
            pltpu.make_async_copy(
                buf.at[s], out_hbm.at[pl.ds(row0 + c * _CH, _CH)], osems[s]
            ).wait()

        def pair(i, carry):
            c0, c1 = 2 * i, 2 * i + 1

            @pl.when(i >= 1)
            def _():
                wait_out(c0 - 2, 0)

            h0 = start_gather(c0, 0)

            @pl.when(i >= 1)
            def _():
                wait_out(c1 - 2, 1)

            h1 = start_gather(c1, 1)
            for h in h0:
                h.wait()
            start_out(c0, 0)
            for h in h1:
                h.wait()
            start_out(c1, 1)
            return carry

        lax.fori_loop(0, _NCH // 2, pair, 0)
        wait_out(_NCH - 2, 0)
        wait_out(_NCH - 1, 1)

    return emb


_emb = _build()


@jax.jit
def kernel(input_texts, embedding_table):
    idx = input_texts.reshape(_NW, _G, _K)
    out = _emb(idx, embedding_table)
    return out.reshape(_BATCH, _SEQ, _D)
